# baseline (speedup 1.0000x reference)
import functools

import jax
import jax.numpy as jnp
from jax import lax
from jax.experimental import pallas as pl
from jax.experimental.pallas import tpu as pltpu
from jax.experimental.pallas import tpu_sc as plsc

_N = 10000
_E = 320000
_D = 128
_H = 128
_C = 64

_NC = 2
_NS = 16
_NW = _NC * _NS
_EPW = _E // _NW
_CHUNK = 125
_NCH = _EPW // _CHUNK
_HCH = _NCH // 2

_NP = 10240
_RPT = _NP // _NS
_OCH = 80
_NOCH = _RPT // _OCH

_BR = 1024
_GB = _NP // _BR


def _head_body(pa_ref, pb_ref, w1_ref, b1_ref, wd_ref, bd_ref, o_ref):
    agg = pa_ref[0] + pb_ref[0]
    h = jax.nn.relu(jnp.dot(agg, w1_ref[...],
                            preferred_element_type=jnp.float32) + b1_ref[...])
    o = jnp.dot(h, wd_ref[...],
                preferred_element_type=jnp.float32) + bd_ref[...]
    o_ref[...] = o.T


_sc_mesh = plsc.VectorSubcoreMesh(core_axis_name="c", subcore_axis_name="s")


@functools.partial(
    pl.kernel,
    mesh=_sc_mesh,
    out_type=jax.ShapeDtypeStruct((_NC, _NP, _D), jnp.float32),
    scratch_types=[
        pltpu.VMEM((_HCH, _CHUNK), jnp.int32),
        pltpu.VMEM((_HCH, _CHUNK), jnp.int32),
        pltpu.VMEM((2, _CHUNK, _D), jnp.float32),
        pltpu.VMEM_SHARED((_NP, _D), jnp.float32),
        pltpu.SemaphoreType.DMA,
        pltpu.SemaphoreType.DMA,
    ],
)
def _sc_agg(x_hbm, edges_hbm, out_hbm,
            src_v, dst_v, rows_v, acc_sh, semg, sems):
    cid = lax.axis_index("c")
    sid = lax.axis_index("s")
    wid = sid * _NC + cid

    def _zero_row(r, carry):
        for c in range(_D // 16):
            rows_v[0, r, pl.ds(c * 16, 16)] = jnp.zeros((16,), jnp.float32)
        return carry
    lax.fori_loop(0, _OCH, _zero_row, 0)
    for t in range(_NOCH):
        pltpu.async_copy(rows_v.at[0, pl.ds(0, _OCH)],
                         acc_sh.at[pl.ds(sid * _RPT + t * _OCH, _OCH)], sems)
    for t in range(_NOCH):
        pltpu.make_async_copy(rows_v.at[0, pl.ds(0, _OCH)],
                              acc_sh.at[pl.ds(sid * _RPT, _OCH)], sems).wait()
    plsc.subcore_barrier()

    for phase in range(2):
        pltpu.sync_copy(edges_hbm.at[0, wid, pl.ds(phase * _HCH, _HCH)],
                        src_v)
        pltpu.sync_copy(edges_hbm.at[1, wid, pl.ds(phase * _HCH, _HCH)],
                        dst_v)
        pltpu.async_copy(x_hbm.at[src_v.at[0]], rows_v.at[0], semg)
        pltpu.async_copy(x_hbm.at[src_v.at[1]], rows_v.at[1], semg)
        pltpu.make_async_copy(x_hbm.at[src_v.at[0]],
                              rows_v.at[0], semg).wait()
        pltpu.async_copy(rows_v.at[0], acc_sh.at[dst_v.at[0]],
                         sems, add=True)

        def _edge_chunk(j, carry):
            b = lax.rem(j, 2)
            pltpu.make_async_copy(x_hbm.at[src_v.at[j]],
                                  rows_v.at[b], semg).wait()
            pltpu.make_async_copy(rows_v.at[1 - b],
                                  acc_sh.at[dst_v.at[j - 1]], sems).wait()
            pltpu.async_copy(x_hbm.at[src_v.at[j + 1]],
                             rows_v.at[1 - b], semg)
            pltpu.async_copy(rows_v.at[b], acc_sh.at[dst_v.at[j]],
                             sems, add=True)
            return carry
        lax.fori_loop(1, _HCH - 1, _edge_chunk, 0)
        _lb = (_HCH - 1) % 2
        pltpu.make_async_copy(x_hbm.at[src_v.at[_HCH - 1]],
                              rows_v.at[_lb], semg).wait()
        pltpu.make_async_copy(rows_v.at[1 - _lb],
                              acc_sh.at[dst_v.at[_HCH - 2]],
                              sems).wait()
        pltpu.async_copy(rows_v.at[_lb], acc_sh.at[dst_v.at[_HCH - 1]],
                         sems, add=True)
        pltpu.make_async_copy(rows_v.at[_lb],
                              acc_sh.at[dst_v.at[_HCH - 1]],
                              sems).wait()
    plsc.subcore_barrier()

    for t in range(_NOCH):
        base = sid * _RPT + t * _OCH
        pltpu.async_copy(acc_sh.at[pl.ds(base, _OCH)],
                         out_hbm.at[cid, pl.ds(base, _OCH)], semg)
    for t in range(_NOCH):
        pltpu.make_async_copy(acc_sh.at[pl.ds(sid * _RPT, _OCH)],
                              out_hbm.at[cid, pl.ds(sid * _RPT, _OCH)],
                              semg).wait()


def kernel(x, adj, W1, b1, Wd, bd):
    edges = adj.reshape(2, _NW, _NCH, _CHUNK)

    partials = _sc_agg(x, edges)

    out = pl.pallas_call(
        _head_body,
        grid=(_GB,),
        in_specs=[pl.BlockSpec((1, _BR, _D), lambda i: (0, i, 0)),
                  pl.BlockSpec((1, _BR, _D), lambda i: (1, i, 0)),
                  pl.BlockSpec((_D, _H), lambda i: (0, 0)),
                  pl.BlockSpec((1, _H), lambda i: (0, 0)),
                  pl.BlockSpec((_H, _C), lambda i: (0, 0)),
                  pl.BlockSpec((1, _C), lambda i: (0, 0))],
        out_specs=pl.BlockSpec((_C, _BR), lambda i: (0, i)),
        out_shape=jax.ShapeDtypeStruct((_C, _NP), jnp.float32),
    )(partials, partials, W1, b1[None], Wd, bd[None])
    return out[:, :_N].T


# speedup vs baseline: 1.0088x; 1.0088x over previous
import functools

import jax
import jax.numpy as jnp
from jax import lax
from jax.experimental import pallas as pl
from jax.experimental.pallas import tpu as pltpu
from jax.experimental.pallas import tpu_sc as plsc

_N = 10000
_E = 320000
_D = 128
_H = 128
_C = 64

_NC = 2
_NS = 16
_NW = _NC * _NS
_EPW = _E // _NW
_CHUNK = 125
_NCH = _EPW // _CHUNK
_HCH = _NCH // 2

_NP = 10240
_RPT = _NP // _NS
_OCH = 80
_NOCH = _RPT // _OCH

_BR = 1024
_GB = _NP // _BR


def _head_body(pa_ref, pb_ref, w1_ref, b1_ref, wd_ref, bd_ref, o_ref):
    agg = pa_ref[0] + pb_ref[0]
    h = jax.nn.relu(jnp.dot(agg, w1_ref[...],
                            preferred_element_type=jnp.float32) + b1_ref[...])
    o = jnp.dot(h, wd_ref[...],
                preferred_element_type=jnp.float32) + bd_ref[...]
    o_ref[...] = o.T


_sc_mesh = plsc.VectorSubcoreMesh(core_axis_name="c", subcore_axis_name="s")


@functools.partial(
    pl.kernel,
    mesh=_sc_mesh,
    out_type=jax.ShapeDtypeStruct((_NC, _NP, _D), jnp.float32),
    scratch_types=[
        pltpu.VMEM((_HCH, _CHUNK), jnp.int32),
        pltpu.VMEM((_HCH, _CHUNK), jnp.int32),
        pltpu.VMEM((2, _CHUNK, _D), jnp.float32),
        pltpu.VMEM_SHARED((_NP, _D), jnp.float32),
        pltpu.SemaphoreType.DMA,
        pltpu.SemaphoreType.DMA,
    ],
)
def _sc_agg(x_hbm, edges_hbm, out_hbm,
            src_v, dst_v, rows_v, acc_sh, semg, sems):
    cid = lax.axis_index("c")
    sid = lax.axis_index("s")
    wid = sid * _NC + cid

    pltpu.sync_copy(edges_hbm.at[0, wid, pl.ds(0, _HCH)], src_v)
    pltpu.sync_copy(edges_hbm.at[1, wid, pl.ds(0, _HCH)], dst_v)
    pltpu.async_copy(x_hbm.at[src_v.at[0]], rows_v.at[0], semg)

    def _zero_row(r, carry):
        for c in range(_D // 16):
            rows_v[1, r, pl.ds(c * 16, 16)] = jnp.zeros((16,), jnp.float32)
        return carry
    lax.fori_loop(0, _OCH, _zero_row, 0)
    for t in range(_NOCH):
        pltpu.async_copy(rows_v.at[1, pl.ds(0, _OCH)],
                         acc_sh.at[pl.ds(sid * _RPT + t * _OCH, _OCH)], sems)
    for t in range(_NOCH):
        pltpu.make_async_copy(rows_v.at[1, pl.ds(0, _OCH)],
                              acc_sh.at[pl.ds(sid * _RPT, _OCH)], sems).wait()
    plsc.subcore_barrier()

    for phase in range(2):
        if phase:
            pltpu.sync_copy(edges_hbm.at[0, wid, pl.ds(phase * _HCH, _HCH)],
                            src_v)
            pltpu.sync_copy(edges_hbm.at[1, wid, pl.ds(phase * _HCH, _HCH)],
                            dst_v)
            pltpu.async_copy(x_hbm.at[src_v.at[0]], rows_v.at[0], semg)
        pltpu.async_copy(x_hbm.at[src_v.at[1]], rows_v.at[1], semg)
        pltpu.make_async_copy(x_hbm.at[src_v.at[0]],
                              rows_v.at[0], semg).wait()
        pltpu.async_copy(rows_v.at[0], acc_sh.at[dst_v.at[0]],
                         sems, add=True)

        def _edge_chunk(j, carry):
            b = lax.rem(j, 2)
            pltpu.make_async_copy(x_hbm.at[src_v.at[j]],
                                  rows_v.at[b], semg).wait()
            pltpu.make_async_copy(rows_v.at[1 - b],
                                  acc_sh.at[dst_v.at[j - 1]], sems).wait()
            pltpu.async_copy(x_hbm.at[src_v.at[j + 1]],
                             rows_v.at[1 - b], semg)
            pltpu.async_copy(rows_v.at[b], acc_sh.at[dst_v.at[j]],
                             sems, add=True)
            return carry
        lax.fori_loop(1, _HCH - 1, _edge_chunk, 0)
        _lb = (_HCH - 1) % 2
        pltpu.make_async_copy(x_hbm.at[src_v.at[_HCH - 1]],
                              rows_v.at[_lb], semg).wait()
        pltpu.make_async_copy(rows_v.at[1 - _lb],
                              acc_sh.at[dst_v.at[_HCH - 2]],
                              sems).wait()
        pltpu.async_copy(rows_v.at[_lb], acc_sh.at[dst_v.at[_HCH - 1]],
                         sems, add=True)
        pltpu.make_async_copy(rows_v.at[_lb],
                              acc_sh.at[dst_v.at[_HCH - 1]],
                              sems).wait()
    plsc.subcore_barrier()

    for t in range(_NOCH):
        base = sid * _RPT + t * _OCH
        pltpu.async_copy(acc_sh.at[pl.ds(base, _OCH)],
                         out_hbm.at[cid, pl.ds(base, _OCH)], semg)
    for t in range(_NOCH):
        pltpu.make_async_copy(acc_sh.at[pl.ds(sid * _RPT, _OCH)],
                              out_hbm.at[cid, pl.ds(sid * _RPT, _OCH)],
                              semg).wait()


def kernel(x, adj, W1, b1, Wd, bd):
    edges = adj.reshape(2, _NW, _NCH, _CHUNK)

    partials = _sc_agg(x, edges)

    out = pl.pallas_call(
        _head_body,
        grid=(_GB,),
        in_specs=[pl.BlockSpec((1, _BR, _D), lambda i: (0, i, 0)),
                  pl.BlockSpec((1, _BR, _D), lambda i: (1, i, 0)),
                  pl.BlockSpec((_D, _H), lambda i: (0, 0)),
                  pl.BlockSpec((1, _H), lambda i: (0, 0)),
                  pl.BlockSpec((_H, _C), lambda i: (0, 0)),
                  pl.BlockSpec((1, _C), lambda i: (0, 0))],
        out_specs=pl.BlockSpec((_C, _BR), lambda i: (0, i)),
        out_shape=jax.ShapeDtypeStruct((_C, _NP), jnp.float32),
    )(partials, partials, W1, b1[None], Wd, bd[None])
    return out[:, :_N].T
